# baseline (speedup 1.0000x reference)
import jax
import jax.numpy as jnp
from jax import lax
from jax.experimental import pallas as pl
from jax.experimental.pallas import tpu as pltpu
from jax.experimental.pallas import tpu_sc as plsc

Z = 0.8
NC, NS = 2, 16
NW = NC * NS
K = 176
R = 400
D = 128
DH = D // 2


def _pre_body(x_ref, m_ref, w0t_ref, b0_ref, w1t_ref, b1_ref,
              out0_ref, out1_ref):
    x = x_ref[...]
    x0 = jnp.maximum(
        jnp.dot(x, w0t_ref[...], preferred_element_type=jnp.float32,
                precision=lax.Precision.HIGHEST) + b0_ref[...], 0.0)
    x1 = jnp.maximum(
        jnp.dot(x, w1t_ref[...], preferred_element_type=jnp.float32,
                precision=lax.Precision.HIGHEST) + b1_ref[...], 0.0)
    m = m_ref[...]
    a = Z * x1 + (1.0 - Z) * x0
    b = Z * x0 + (1.0 - Z) * x1
    res = b + m * (a - b)
    out0_ref[...] = res[:, :DH]
    out1_ref[...] = res[:, DH:]


def _sc_body(src3_hbm, dst3_hbm, w3_hbm, xb0_hbm, xb1_hbm, zx_hbm, zw_hbm,
             px0_hbm, px1_hbm, pw_hbm,
             src_a, dst_a, w_a, r0, r1, r2, wr0, wr1, wr2,
             g0, g1, g2, s0, s1, s2, w0, w1, w2, acc_x, acc_w):
    rows = (r0, r1, r2)
    wrow = (wr0, wr1, wr2)
    gsem = (g0, g1, g2)
    ssem = (s0, s1, s2)
    wsem = (w0, w1, w2)
    NB = 3
    npad = acc_x.shape[0]
    nchunk = src_a.shape[0]
    c = lax.axis_index("c")
    s = lax.axis_index("s")
    wid = c * NS + s
    rpt = npad // NS
    lane0 = jnp.where(lax.iota(jnp.int32, 16) == 0, 1.0, 0.0)
    gdn = lax.GatherDimensionNumbers(
        offset_dims=(), collapsed_slice_dims=(0,), start_index_map=(0,))

    pltpu.sync_copy(src3_hbm.at[wid], src_a)
    pltpu.sync_copy(dst3_hbm.at[wid], dst_a)
    pltpu.sync_copy(w3_hbm.at[wid], w_a)

    for h, xb_hbm, px_hbm in ((0, xb0_hbm, px0_hbm), (1, xb1_hbm, px1_hbm)):
        pltpu.sync_copy(zx_hbm, acc_x.at[pl.ds(s * rpt, rpt)])
        if h == 0:
            pltpu.sync_copy(zw_hbm, acc_w.at[pl.ds(s * rpt, rpt)])
        plsc.subcore_barrier()

        def scale_chunk(j, rb, wb_ref):
            @plsc.parallel_loop(0, K // 16, 1, unroll=K // 16)
            def escale(g):
                wg = w_a[j, pl.ds(g * 16, 16)]
                for l in range(16):
                    idx = (jnp.zeros((16,), jnp.int32) + l).reshape(16, 1)
                    wb = lax.gather(wg, idx, gdn, (1,),
                                    mode=lax.GatherScatterMode.PROMISE_IN_BOUNDS)
                    e = g * 16 + l
                    if h == 0:
                        wb_ref[e, pl.ds(0, 16)] = wb * lane0
                    for d in range(DH // 16):
                        rb[e, pl.ds(d * 16, 16)] = rb[e, pl.ds(d * 16, 16)] * wb

        def iter_one(j, b, wait_pred, gather_j2, gather_pred=None):
            bp = (b + 2) % NB
            pltpu.make_async_copy(xb_hbm.at[dst_a.at[j]], rows[b],
                                  gsem[b]).wait()
            scale_chunk(j, rows[b], wrow[b])

            def retire():
                pltpu.make_async_copy(rows[bp], acc_x.at[src_a.at[j]],
                                      ssem[bp]).wait()
                if h == 0:
                    pltpu.make_async_copy(wrow[bp], acc_w.at[src_a.at[j]],
                                          wsem[bp]).wait()
            if wait_pred is None:
                retire()
            else:
                pl.when(wait_pred)(retire)
            if gather_j2 is not None:
                def launch():
                    pltpu.async_copy(xb_hbm.at[dst_a.at[gather_j2]], rows[bp],
                                     gsem[bp])
                if gather_pred is None:
                    launch()
                else:
                    pl.when(gather_pred)(launch)
            pltpu.async_copy(rows[b], acc_x.at[src_a.at[j]], ssem[b], add=True)
            if h == 0:
                pltpu.async_copy(wrow[b], acc_w.at[src_a.at[j]], wsem[b],
                                 add=True)

        pltpu.async_copy(xb_hbm.at[dst_a.at[0]], rows[0], gsem[0])
        pltpu.async_copy(xb_hbm.at[dst_a.at[1]], rows[1], gsem[1])

        nloop = nchunk // 3

        def body(i, carry):
            j0 = 3 * i
            iter_one(j0, 0, i > 0, j0 + 2, j0 + 2 < nchunk)
            iter_one(j0 + 1, 1, None, j0 + 3, j0 + 3 < nchunk)
            iter_one(j0 + 2, 2, None, j0 + 4, j0 + 4 < nchunk)
            return carry
        lax.fori_loop(0, nloop, body, 0)
        for jj in range(3 * nloop, nchunk):
            iter_one(jj, jj % 3, None, None)
        blast = (nchunk - 1) % 3
        pltpu.make_async_copy(rows[blast], acc_x.at[src_a.at[nchunk - 1]],
                              ssem[blast]).wait()
        if h == 0:
            pltpu.make_async_copy(wrow[blast], acc_w.at[src_a.at[nchunk - 1]],
                                  wsem[blast]).wait()

        plsc.subcore_barrier()
        pltpu.sync_copy(acc_x.at[pl.ds(s * rpt, rpt)],
                        px_hbm.at[c, pl.ds(s * rpt, rpt)])
        if h == 0:
            pltpu.sync_copy(acc_w.at[pl.ds(s * rpt, rpt)],
                            pw_hbm.at[c, pl.ds(s * rpt, rpt)])


def _post_body(px0_ref, px1_ref, pw_ref, x_ref, m_ref,
               a0_ref, c0_ref, a1_ref, c1_ref,
               bias0_ref, bias1_ref, g_ref, be_ref, out_ref):
    accx = jnp.concatenate(
        [px0_ref[0] + px0_ref[1], px1_ref[0] + px1_ref[1]], axis=1)
    deg = pw_ref[0, :, 0:1] + pw_ref[1, :, 0:1]
    deg = jnp.where(deg < 0.5, deg + 1.0, deg)
    xm = accx / deg
    mu = jnp.mean(xm, axis=1, keepdims=True)
    var = jnp.mean((xm - mu) * (xm - mu), axis=1, keepdims=True)
    xn = (xm - mu) * lax.rsqrt(var + 1e-5) * g_ref[...] + be_ref[...]
    xin = x_ref[...]
    hi = lax.Precision.HIGHEST
    y0 = (jnp.dot(xn, a0_ref[...], preferred_element_type=jnp.float32, precision=hi)
          + jnp.dot(xin, c0_ref[...], preferred_element_type=jnp.float32, precision=hi)
          + bias0_ref[...])
    y1 = (jnp.dot(xn, a1_ref[...], preferred_element_type=jnp.float32, precision=hi)
          + jnp.dot(xin, c1_ref[...], preferred_element_type=jnp.float32, precision=hi)
          + bias1_ref[...])
    m = m_ref[...]
    a = Z * y1 + (1.0 - Z) * y0
    b = Z * y0 + (1.0 - Z) * y1
    out_ref[...] = b + m * (a - b)


def kernel(x_, edge_index, edge_weight, mask, W_t0, b_t0, W_t1, b_t1,
           W_c0, b_c0, W_c1, b_c1, gamma, beta):
    n, d_in = x_.shape
    e_total = edge_weight.shape[0]
    src = edge_index[0].astype(jnp.int32)
    dst = edge_index[1].astype(jnp.int32)
    w = edge_weight.astype(jnp.float32)
    m = mask.astype(jnp.float32)
    grid = n // R

    xb0, xb1 = pl.pallas_call(
        _pre_body,
        grid=(grid,),
        in_specs=[
            pl.BlockSpec((R, D), lambda i: (i, 0)),
            pl.BlockSpec((R, 1), lambda i: (i, 0)),
            pl.BlockSpec((D, D), lambda i: (0, 0)),
            pl.BlockSpec((1, D), lambda i: (0, 0)),
            pl.BlockSpec((D, D), lambda i: (0, 0)),
            pl.BlockSpec((1, D), lambda i: (0, 0)),
        ],
        out_specs=[pl.BlockSpec((R, DH), lambda i: (i, 0)),
                   pl.BlockSpec((R, DH), lambda i: (i, 0))],
        out_shape=[jax.ShapeDtypeStruct((n, DH), jnp.float32),
                   jax.ShapeDtypeStruct((n, DH), jnp.float32)],
    )(x_, m, W_t0.T, b_t0.reshape(1, D), W_t1.T, b_t1.reshape(1, D))

    npad = ((n // NS + 7) // 8 * 8) * NS
    rpt = npad // NS
    epw_raw = e_total // NW
    nchunk = -(-epw_raw // K)
    pad = NW * nchunk * K - e_total
    if pad:
        src = jnp.concatenate([src, jnp.full((pad,), n, jnp.int32)])
        dst = jnp.concatenate([dst, jnp.zeros((pad,), jnp.int32)])
        w = jnp.concatenate([w, jnp.zeros((pad,), jnp.float32)])
    zx = jnp.zeros((rpt, DH), jnp.float32)
    zw = jnp.zeros((rpt, 16), jnp.float32)
    src3 = src.reshape(NW, nchunk, K)
    dst3 = dst.reshape(NW, nchunk, K)
    w3 = w.reshape(NW, nchunk, K)
    px0, px1, pw = pl.kernel(
        _sc_body,
        out_type=[
            jax.ShapeDtypeStruct((NC, npad, DH), jnp.float32),
            jax.ShapeDtypeStruct((NC, npad, DH), jnp.float32),
            jax.ShapeDtypeStruct((NC, npad, 16), jnp.float32),
        ],
        mesh=plsc.VectorSubcoreMesh(core_axis_name="c", subcore_axis_name="s"),
        compiler_params=pltpu.CompilerParams(use_tc_tiling_on_sc=False),
        scratch_types=[
            pltpu.VMEM((nchunk, K), jnp.int32),
            pltpu.VMEM((nchunk, K), jnp.int32),
            pltpu.VMEM((nchunk, K), jnp.float32),
            pltpu.VMEM((K, DH), jnp.float32),
            pltpu.VMEM((K, DH), jnp.float32),
            pltpu.VMEM((K, DH), jnp.float32),
            pltpu.VMEM((K, 16), jnp.float32),
            pltpu.VMEM((K, 16), jnp.float32),
            pltpu.VMEM((K, 16), jnp.float32),
            pltpu.SemaphoreType.DMA,
            pltpu.SemaphoreType.DMA,
            pltpu.SemaphoreType.DMA,
            pltpu.SemaphoreType.DMA,
            pltpu.SemaphoreType.DMA,
            pltpu.SemaphoreType.DMA,
            pltpu.SemaphoreType.DMA,
            pltpu.SemaphoreType.DMA,
            pltpu.SemaphoreType.DMA,
            pltpu.VMEM_SHARED((npad, DH), jnp.float32),
            pltpu.VMEM_SHARED((npad, 16), jnp.float32),
        ],
    )(src3, dst3, w3, xb0, xb1, zx, zw)

    out = pl.pallas_call(
        _post_body,
        grid=(grid,),
        in_specs=[
            pl.BlockSpec((NC, R, DH), lambda i: (0, i, 0)),
            pl.BlockSpec((NC, R, DH), lambda i: (0, i, 0)),
            pl.BlockSpec((NC, R, 16), lambda i: (0, i, 0)),
            pl.BlockSpec((R, D), lambda i: (i, 0)),
            pl.BlockSpec((R, 1), lambda i: (i, 0)),
            pl.BlockSpec((D, D), lambda i: (0, 0)),
            pl.BlockSpec((D, D), lambda i: (0, 0)),
            pl.BlockSpec((D, D), lambda i: (0, 0)),
            pl.BlockSpec((D, D), lambda i: (0, 0)),
            pl.BlockSpec((1, D), lambda i: (0, 0)),
            pl.BlockSpec((1, D), lambda i: (0, 0)),
            pl.BlockSpec((1, D), lambda i: (0, 0)),
            pl.BlockSpec((1, D), lambda i: (0, 0)),
        ],
        out_specs=pl.BlockSpec((R, D), lambda i: (i, 0)),
        out_shape=jax.ShapeDtypeStruct((n, D), jnp.float32),
    )(px0, px1, pw, x_, m,
      W_c0[:, :D].T, W_c0[:, D:].T, W_c1[:, :D].T, W_c1[:, D:].T,
      b_c0.reshape(1, D), b_c1.reshape(1, D),
      gamma.reshape(1, D), beta.reshape(1, D))
    return out


# speedup vs baseline: 1.1376x; 1.1376x over previous
import jax
import jax.numpy as jnp
from jax import lax
from jax.experimental import pallas as pl
from jax.experimental.pallas import tpu as pltpu
from jax.experimental.pallas import tpu_sc as plsc

Z = 0.8
NC, NS = 2, 16
NW = NC * NS
K = 80
R = 400
D = 128
DH = D // 2


def _pre_body(x_ref, m_ref, w0t_ref, b0_ref, w1t_ref, b1_ref,
              out0_ref, out1_ref):
    x = x_ref[...]
    x0 = jnp.maximum(
        jnp.dot(x, w0t_ref[...], preferred_element_type=jnp.float32,
                precision=lax.Precision.HIGHEST) + b0_ref[...], 0.0)
    x1 = jnp.maximum(
        jnp.dot(x, w1t_ref[...], preferred_element_type=jnp.float32,
                precision=lax.Precision.HIGHEST) + b1_ref[...], 0.0)
    m = m_ref[...]
    a = Z * x1 + (1.0 - Z) * x0
    b = Z * x0 + (1.0 - Z) * x1
    res = b + m * (a - b)
    out0_ref[...] = res[:, :DH]
    out1_ref[...] = res[:, DH:]


def _sc_body(src3_hbm, dst3_hbm, w3_hbm, xb0_hbm, xb1_hbm, zx_hbm, zw_hbm,
             px0_hbm, px1_hbm, pw_hbm,
             src_a, dst_a, w_a, r0, r1, r2, wr0, wr1, wr2,
             g0, g1, g2, s0, s1, s2, w0, w1, w2, acc_x, acc_w):
    rows = (r0, r1, r2)
    wrow = (wr0, wr1, wr2)
    gsem = (g0, g1, g2)
    ssem = (s0, s1, s2)
    wsem = (w0, w1, w2)
    NB = 3
    npad = acc_x.shape[0]
    nchunk = src_a.shape[0]
    c = lax.axis_index("c")
    s = lax.axis_index("s")
    wid = c * NS + s
    rpt = npad // NS
    lane0 = jnp.where(lax.iota(jnp.int32, 16) == 0, 1.0, 0.0)
    gdn = lax.GatherDimensionNumbers(
        offset_dims=(), collapsed_slice_dims=(0,), start_index_map=(0,))

    pltpu.sync_copy(src3_hbm.at[wid], src_a)
    pltpu.sync_copy(dst3_hbm.at[wid], dst_a)
    pltpu.sync_copy(w3_hbm.at[wid], w_a)

    for h, xb_hbm, px_hbm in ((0, xb0_hbm, px0_hbm), (1, xb1_hbm, px1_hbm)):
        pltpu.sync_copy(zx_hbm, acc_x.at[pl.ds(s * rpt, rpt)])
        if h == 0:
            pltpu.sync_copy(zw_hbm, acc_w.at[pl.ds(s * rpt, rpt)])
        plsc.subcore_barrier()

        def scale_chunk(j, rb, wb_ref):
            @plsc.parallel_loop(0, K // 16, 1, unroll=K // 16)
            def escale(g):
                wg = w_a[j, pl.ds(g * 16, 16)]
                for l in range(16):
                    idx = (jnp.zeros((16,), jnp.int32) + l).reshape(16, 1)
                    wb = lax.gather(wg, idx, gdn, (1,),
                                    mode=lax.GatherScatterMode.PROMISE_IN_BOUNDS)
                    e = g * 16 + l
                    if h == 0:
                        wb_ref[e, pl.ds(0, 16)] = wb * lane0
                    for d in range(DH // 16):
                        rb[e, pl.ds(d * 16, 16)] = rb[e, pl.ds(d * 16, 16)] * wb

        def iter_one(j, b, wait_pred, gather_j2, gather_pred=None):
            bp = (b + 2) % NB
            pltpu.make_async_copy(xb_hbm.at[dst_a.at[j]], rows[b],
                                  gsem[b]).wait()
            scale_chunk(j, rows[b], wrow[b])

            def retire():
                pltpu.make_async_copy(rows[bp], acc_x.at[src_a.at[j]],
                                      ssem[bp]).wait()
                if h == 0:
                    pltpu.make_async_copy(wrow[bp], acc_w.at[src_a.at[j]],
                                          wsem[bp]).wait()
            if wait_pred is None:
                retire()
            else:
                pl.when(wait_pred)(retire)
            if gather_j2 is not None:
                def launch():
                    pltpu.async_copy(xb_hbm.at[dst_a.at[gather_j2]], rows[bp],
                                     gsem[bp])
                if gather_pred is None:
                    launch()
                else:
                    pl.when(gather_pred)(launch)
            pltpu.async_copy(rows[b], acc_x.at[src_a.at[j]], ssem[b], add=True)
            if h == 0:
                pltpu.async_copy(wrow[b], acc_w.at[src_a.at[j]], wsem[b],
                                 add=True)

        pltpu.async_copy(xb_hbm.at[dst_a.at[0]], rows[0], gsem[0])
        pltpu.async_copy(xb_hbm.at[dst_a.at[1]], rows[1], gsem[1])

        nloop = nchunk // 3

        def body(i, carry):
            j0 = 3 * i
            iter_one(j0, 0, i > 0, j0 + 2, j0 + 2 < nchunk)
            iter_one(j0 + 1, 1, None, j0 + 3, j0 + 3 < nchunk)
            iter_one(j0 + 2, 2, None, j0 + 4, j0 + 4 < nchunk)
            return carry
        lax.fori_loop(0, nloop, body, 0)
        for jj in range(3 * nloop, nchunk):
            iter_one(jj, jj % 3, None, None)
        blast = (nchunk - 1) % 3
        pltpu.make_async_copy(rows[blast], acc_x.at[src_a.at[nchunk - 1]],
                              ssem[blast]).wait()
        if h == 0:
            pltpu.make_async_copy(wrow[blast], acc_w.at[src_a.at[nchunk - 1]],
                                  wsem[blast]).wait()

        plsc.subcore_barrier()
        pltpu.sync_copy(acc_x.at[pl.ds(s * rpt, rpt)],
                        px_hbm.at[c, pl.ds(s * rpt, rpt)])
        if h == 0:
            pltpu.sync_copy(acc_w.at[pl.ds(s * rpt, rpt)],
                            pw_hbm.at[c, pl.ds(s * rpt, rpt)])


def _post_body(px0_ref, px1_ref, pw_ref, x_ref, m_ref,
               a0_ref, c0_ref, a1_ref, c1_ref,
               bias0_ref, bias1_ref, g_ref, be_ref, out_ref):
    accx = jnp.concatenate(
        [px0_ref[0] + px0_ref[1], px1_ref[0] + px1_ref[1]], axis=1)
    deg = pw_ref[0, :, 0:1] + pw_ref[1, :, 0:1]
    deg = jnp.where(deg < 0.5, deg + 1.0, deg)
    xm = accx / deg
    mu = jnp.mean(xm, axis=1, keepdims=True)
    var = jnp.mean((xm - mu) * (xm - mu), axis=1, keepdims=True)
    xn = (xm - mu) * lax.rsqrt(var + 1e-5) * g_ref[...] + be_ref[...]
    xin = x_ref[...]
    hi = lax.Precision.HIGHEST
    y0 = (jnp.dot(xn, a0_ref[...], preferred_element_type=jnp.float32, precision=hi)
          + jnp.dot(xin, c0_ref[...], preferred_element_type=jnp.float32, precision=hi)
          + bias0_ref[...])
    y1 = (jnp.dot(xn, a1_ref[...], preferred_element_type=jnp.float32, precision=hi)
          + jnp.dot(xin, c1_ref[...], preferred_element_type=jnp.float32, precision=hi)
          + bias1_ref[...])
    m = m_ref[...]
    a = Z * y1 + (1.0 - Z) * y0
    b = Z * y0 + (1.0 - Z) * y1
    out_ref[...] = b + m * (a - b)


def kernel(x_, edge_index, edge_weight, mask, W_t0, b_t0, W_t1, b_t1,
           W_c0, b_c0, W_c1, b_c1, gamma, beta):
    n, d_in = x_.shape
    e_total = edge_weight.shape[0]
    src = edge_index[0].astype(jnp.int32)
    dst = edge_index[1].astype(jnp.int32)
    w = edge_weight.astype(jnp.float32)
    m = mask.astype(jnp.float32)
    grid = n // R

    xb0, xb1 = pl.pallas_call(
        _pre_body,
        grid=(grid,),
        in_specs=[
            pl.BlockSpec((R, D), lambda i: (i, 0)),
            pl.BlockSpec((R, 1), lambda i: (i, 0)),
            pl.BlockSpec((D, D), lambda i: (0, 0)),
            pl.BlockSpec((1, D), lambda i: (0, 0)),
            pl.BlockSpec((D, D), lambda i: (0, 0)),
            pl.BlockSpec((1, D), lambda i: (0, 0)),
        ],
        out_specs=[pl.BlockSpec((R, DH), lambda i: (i, 0)),
                   pl.BlockSpec((R, DH), lambda i: (i, 0))],
        out_shape=[jax.ShapeDtypeStruct((n, DH), jnp.float32),
                   jax.ShapeDtypeStruct((n, DH), jnp.float32)],
    )(x_, m, W_t0.T, b_t0.reshape(1, D), W_t1.T, b_t1.reshape(1, D))

    npad = ((n // NS + 7) // 8 * 8) * NS
    rpt = npad // NS
    epw_raw = e_total // NW
    nchunk = -(-epw_raw // K)
    pad = NW * nchunk * K - e_total
    if pad:
        src = jnp.concatenate([src, jnp.full((pad,), n, jnp.int32)])
        dst = jnp.concatenate([dst, jnp.zeros((pad,), jnp.int32)])
        w = jnp.concatenate([w, jnp.zeros((pad,), jnp.float32)])
    zx = jnp.zeros((rpt, DH), jnp.float32)
    zw = jnp.zeros((rpt, 16), jnp.float32)
    src3 = src.reshape(NW, nchunk, K)
    dst3 = dst.reshape(NW, nchunk, K)
    w3 = w.reshape(NW, nchunk, K)
    px0, px1, pw = pl.kernel(
        _sc_body,
        out_type=[
            jax.ShapeDtypeStruct((NC, npad, DH), jnp.float32),
            jax.ShapeDtypeStruct((NC, npad, DH), jnp.float32),
            jax.ShapeDtypeStruct((NC, npad, 16), jnp.float32),
        ],
        mesh=plsc.VectorSubcoreMesh(core_axis_name="c", subcore_axis_name="s"),
        compiler_params=pltpu.CompilerParams(use_tc_tiling_on_sc=False),
        scratch_types=[
            pltpu.VMEM((nchunk, K), jnp.int32),
            pltpu.VMEM((nchunk, K), jnp.int32),
            pltpu.VMEM((nchunk, K), jnp.float32),
            pltpu.VMEM((K, DH), jnp.float32),
            pltpu.VMEM((K, DH), jnp.float32),
            pltpu.VMEM((K, DH), jnp.float32),
            pltpu.VMEM((K, 16), jnp.float32),
            pltpu.VMEM((K, 16), jnp.float32),
            pltpu.VMEM((K, 16), jnp.float32),
            pltpu.SemaphoreType.DMA,
            pltpu.SemaphoreType.DMA,
            pltpu.SemaphoreType.DMA,
            pltpu.SemaphoreType.DMA,
            pltpu.SemaphoreType.DMA,
            pltpu.SemaphoreType.DMA,
            pltpu.SemaphoreType.DMA,
            pltpu.SemaphoreType.DMA,
            pltpu.SemaphoreType.DMA,
            pltpu.VMEM_SHARED((npad, DH), jnp.float32),
            pltpu.VMEM_SHARED((npad, 16), jnp.float32),
        ],
    )(src3, dst3, w3, xb0, xb1, zx, zw)

    out = pl.pallas_call(
        _post_body,
        grid=(grid,),
        in_specs=[
            pl.BlockSpec((NC, R, DH), lambda i: (0, i, 0)),
            pl.BlockSpec((NC, R, DH), lambda i: (0, i, 0)),
            pl.BlockSpec((NC, R, 16), lambda i: (0, i, 0)),
            pl.BlockSpec((R, D), lambda i: (i, 0)),
            pl.BlockSpec((R, 1), lambda i: (i, 0)),
            pl.BlockSpec((D, D), lambda i: (0, 0)),
            pl.BlockSpec((D, D), lambda i: (0, 0)),
            pl.BlockSpec((D, D), lambda i: (0, 0)),
            pl.BlockSpec((D, D), lambda i: (0, 0)),
            pl.BlockSpec((1, D), lambda i: (0, 0)),
            pl.BlockSpec((1, D), lambda i: (0, 0)),
            pl.BlockSpec((1, D), lambda i: (0, 0)),
            pl.BlockSpec((1, D), lambda i: (0, 0)),
        ],
        out_specs=pl.BlockSpec((R, D), lambda i: (i, 0)),
        out_shape=jax.ShapeDtypeStruct((n, D), jnp.float32),
    )(px0, px1, pw, x_, m,
      W_c0[:, :D].T, W_c0[:, D:].T, W_c1[:, :D].T, W_c1[:, D:].T,
      b_c0.reshape(1, D), b_c1.reshape(1, D),
      gamma.reshape(1, D), beta.reshape(1, D))
    return out


# speedup vs baseline: 1.2634x; 1.1105x over previous
import jax
import jax.numpy as jnp
from jax import lax
from jax.experimental import pallas as pl
from jax.experimental.pallas import tpu as pltpu
from jax.experimental.pallas import tpu_sc as plsc

Z = 0.8
NC, NS = 2, 16
NW = NC * NS
K = 80
R = 400
D = 128
DH = D // 2


def _pre_body(x_ref, m_ref, w0t_ref, b0_ref, w1t_ref, b1_ref,
              out0_ref, out1_ref):
    x = x_ref[...]
    x0 = jnp.maximum(
        jnp.dot(x, w0t_ref[...], preferred_element_type=jnp.float32) + b0_ref[...], 0.0)
    x1 = jnp.maximum(
        jnp.dot(x, w1t_ref[...], preferred_element_type=jnp.float32) + b1_ref[...], 0.0)
    m = m_ref[...]
    a = Z * x1 + (1.0 - Z) * x0
    b = Z * x0 + (1.0 - Z) * x1
    res = b + m * (a - b)
    out0_ref[...] = res[:, :DH]
    out1_ref[...] = res[:, DH:]


def _sc_body(ei4_hbm, w3_hbm, xb0_hbm, xb1_hbm, zx_hbm, zw_hbm,
             px0_hbm, px1_hbm, pw_hbm,
             src_a, dst_a, w_a, r0, r1, r2, wr0, wr1, wr2,
             g0, g1, g2, s0, s1, s2, w0, w1, w2, acc_x, acc_w):
    rows = (r0, r1, r2)
    wrow = (wr0, wr1, wr2)
    gsem = (g0, g1, g2)
    ssem = (s0, s1, s2)
    wsem = (w0, w1, w2)
    NB = 3
    npad = acc_x.shape[0]
    nchunk = src_a.shape[0]
    c = lax.axis_index("c")
    s = lax.axis_index("s")
    wid = c * NS + s
    rpt = npad // NS
    lane0 = jnp.where(lax.iota(jnp.int32, 16) == 0, 1.0, 0.0)
    gdn = lax.GatherDimensionNumbers(
        offset_dims=(), collapsed_slice_dims=(0,), start_index_map=(0,))

    pltpu.sync_copy(ei4_hbm.at[0, wid], src_a)
    pltpu.sync_copy(ei4_hbm.at[1, wid], dst_a)
    pltpu.sync_copy(w3_hbm.at[wid], w_a)

    for h, xb_hbm, px_hbm in ((0, xb0_hbm, px0_hbm), (1, xb1_hbm, px1_hbm)):
        pltpu.sync_copy(zx_hbm, acc_x.at[pl.ds(s * rpt, rpt)])
        if h == 0:
            pltpu.sync_copy(zw_hbm, acc_w.at[pl.ds(s * rpt, rpt)])
        plsc.subcore_barrier()

        def scale_chunk(j, rb, wb_ref):
            @plsc.parallel_loop(0, K // 16, 1, unroll=K // 16)
            def escale(g):
                wg = w_a[j, pl.ds(g * 16, 16)]
                for l in range(16):
                    idx = (jnp.zeros((16,), jnp.int32) + l).reshape(16, 1)
                    wb = lax.gather(wg, idx, gdn, (1,),
                                    mode=lax.GatherScatterMode.PROMISE_IN_BOUNDS)
                    e = g * 16 + l
                    if h == 0:
                        wb_ref[e, pl.ds(0, 16)] = wb * lane0
                    for d in range(DH // 16):
                        rb[e, pl.ds(d * 16, 16)] = rb[e, pl.ds(d * 16, 16)] * wb

        def iter_one(j, b, wait_pred, gather_j2, gather_pred=None):
            bp = (b + 2) % NB
            pltpu.make_async_copy(xb_hbm.at[dst_a.at[j]], rows[b],
                                  gsem[b]).wait()
            scale_chunk(j, rows[b], wrow[b])

            def retire():
                pltpu.make_async_copy(rows[bp], acc_x.at[src_a.at[j]],
                                      ssem[bp]).wait()
                if h == 0:
                    pltpu.make_async_copy(wrow[bp], acc_w.at[src_a.at[j]],
                                          wsem[bp]).wait()
            if wait_pred is None:
                retire()
            else:
                pl.when(wait_pred)(retire)
            if gather_j2 is not None:
                def launch():
                    pltpu.async_copy(xb_hbm.at[dst_a.at[gather_j2]], rows[bp],
                                     gsem[bp])
                if gather_pred is None:
                    launch()
                else:
                    pl.when(gather_pred)(launch)
            pltpu.async_copy(rows[b], acc_x.at[src_a.at[j]], ssem[b], add=True)
            if h == 0:
                pltpu.async_copy(wrow[b], acc_w.at[src_a.at[j]], wsem[b],
                                 add=True)

        pltpu.async_copy(xb_hbm.at[dst_a.at[0]], rows[0], gsem[0])
        pltpu.async_copy(xb_hbm.at[dst_a.at[1]], rows[1], gsem[1])

        nloop = nchunk // 3

        def body(i, carry):
            j0 = 3 * i
            iter_one(j0, 0, i > 0, j0 + 2, j0 + 2 < nchunk)
            iter_one(j0 + 1, 1, None, j0 + 3, j0 + 3 < nchunk)
            iter_one(j0 + 2, 2, None, j0 + 4, j0 + 4 < nchunk)
            return carry
        lax.fori_loop(0, nloop, body, 0)
        for jj in range(3 * nloop, nchunk):
            iter_one(jj, jj % 3, None, None)
        blast = (nchunk - 1) % 3
        pltpu.make_async_copy(rows[blast], acc_x.at[src_a.at[nchunk - 1]],
                              ssem[blast]).wait()
        if h == 0:
            pltpu.make_async_copy(wrow[blast], acc_w.at[src_a.at[nchunk - 1]],
                                  wsem[blast]).wait()

        plsc.subcore_barrier()
        pltpu.sync_copy(acc_x.at[pl.ds(s * rpt, rpt)],
                        px_hbm.at[c, pl.ds(s * rpt, rpt)])
        if h == 0:
            pltpu.sync_copy(acc_w.at[pl.ds(s * rpt, rpt)],
                            pw_hbm.at[c, pl.ds(s * rpt, rpt)])


def _post_body(px0_ref, px1_ref, pw_ref, x_ref, m_ref,
               a0_ref, c0_ref, a1_ref, c1_ref,
               bias0_ref, bias1_ref, g_ref, be_ref, out_ref):
    accx = jnp.concatenate(
        [px0_ref[0] + px0_ref[1], px1_ref[0] + px1_ref[1]], axis=1)
    deg = pw_ref[0, :, 0:1] + pw_ref[1, :, 0:1]
    deg = jnp.where(deg < 0.5, deg + 1.0, deg)
    xm = accx / deg
    mu = jnp.mean(xm, axis=1, keepdims=True)
    var = jnp.mean((xm - mu) * (xm - mu), axis=1, keepdims=True)
    xn = (xm - mu) * lax.rsqrt(var + 1e-5) * g_ref[...] + be_ref[...]
    xin = x_ref[...]
    y0 = (jnp.dot(xn, a0_ref[...], preferred_element_type=jnp.float32)
          + jnp.dot(xin, c0_ref[...], preferred_element_type=jnp.float32)
          + bias0_ref[...])
    y1 = (jnp.dot(xn, a1_ref[...], preferred_element_type=jnp.float32)
          + jnp.dot(xin, c1_ref[...], preferred_element_type=jnp.float32)
          + bias1_ref[...])
    m = m_ref[...]
    a = Z * y1 + (1.0 - Z) * y0
    b = Z * y0 + (1.0 - Z) * y1
    out_ref[...] = b + m * (a - b)


def kernel(x_, edge_index, edge_weight, mask, W_t0, b_t0, W_t1, b_t1,
           W_c0, b_c0, W_c1, b_c1, gamma, beta):
    n, d_in = x_.shape
    e_total = edge_weight.shape[0]
    ei = edge_index.astype(jnp.int32)
    w = edge_weight.astype(jnp.float32)
    m = mask.astype(jnp.float32)
    grid = n // R

    xb0, xb1 = pl.pallas_call(
        _pre_body,
        grid=(grid,),
        in_specs=[
            pl.BlockSpec((R, D), lambda i: (i, 0)),
            pl.BlockSpec((R, 1), lambda i: (i, 0)),
            pl.BlockSpec((D, D), lambda i: (0, 0)),
            pl.BlockSpec((1, D), lambda i: (0, 0)),
            pl.BlockSpec((D, D), lambda i: (0, 0)),
            pl.BlockSpec((1, D), lambda i: (0, 0)),
        ],
        out_specs=[pl.BlockSpec((R, DH), lambda i: (i, 0)),
                   pl.BlockSpec((R, DH), lambda i: (i, 0))],
        out_shape=[jax.ShapeDtypeStruct((n, DH), jnp.float32),
                   jax.ShapeDtypeStruct((n, DH), jnp.float32)],
    )(x_, m, W_t0.T, b_t0.reshape(1, D), W_t1.T, b_t1.reshape(1, D))

    npad = ((n // NS + 7) // 8 * 8) * NS
    rpt = npad // NS
    epw_raw = e_total // NW
    nchunk = -(-epw_raw // K)
    pad = NW * nchunk * K - e_total
    if pad:
        pe = jnp.concatenate([jnp.full((1, pad), n, jnp.int32),
                              jnp.zeros((1, pad), jnp.int32)])
        ei = jnp.concatenate([ei, pe], axis=1)
        w = jnp.concatenate([w, jnp.zeros((pad,), jnp.float32)])
    zx = jnp.zeros((rpt, DH), jnp.float32)
    zw = jnp.zeros((rpt, 16), jnp.float32)
    ei4 = ei.reshape(2, NW, nchunk, K)
    w3 = w.reshape(NW, nchunk, K)
    px0, px1, pw = pl.kernel(
        _sc_body,
        out_type=[
            jax.ShapeDtypeStruct((NC, npad, DH), jnp.float32),
            jax.ShapeDtypeStruct((NC, npad, DH), jnp.float32),
            jax.ShapeDtypeStruct((NC, npad, 16), jnp.float32),
        ],
        mesh=plsc.VectorSubcoreMesh(core_axis_name="c", subcore_axis_name="s"),
        compiler_params=pltpu.CompilerParams(use_tc_tiling_on_sc=False),
        scratch_types=[
            pltpu.VMEM((nchunk, K), jnp.int32),
            pltpu.VMEM((nchunk, K), jnp.int32),
            pltpu.VMEM((nchunk, K), jnp.float32),
            pltpu.VMEM((K, DH), jnp.float32),
            pltpu.VMEM((K, DH), jnp.float32),
            pltpu.VMEM((K, DH), jnp.float32),
            pltpu.VMEM((K, 16), jnp.float32),
            pltpu.VMEM((K, 16), jnp.float32),
            pltpu.VMEM((K, 16), jnp.float32),
            pltpu.SemaphoreType.DMA,
            pltpu.SemaphoreType.DMA,
            pltpu.SemaphoreType.DMA,
            pltpu.SemaphoreType.DMA,
            pltpu.SemaphoreType.DMA,
            pltpu.SemaphoreType.DMA,
            pltpu.SemaphoreType.DMA,
            pltpu.SemaphoreType.DMA,
            pltpu.SemaphoreType.DMA,
            pltpu.VMEM_SHARED((npad, DH), jnp.float32),
            pltpu.VMEM_SHARED((npad, 16), jnp.float32),
        ],
    )(ei4, w3, xb0, xb1, zx, zw)

    out = pl.pallas_call(
        _post_body,
        grid=(grid,),
        in_specs=[
            pl.BlockSpec((NC, R, DH), lambda i: (0, i, 0)),
            pl.BlockSpec((NC, R, DH), lambda i: (0, i, 0)),
            pl.BlockSpec((NC, R, 16), lambda i: (0, i, 0)),
            pl.BlockSpec((R, D), lambda i: (i, 0)),
            pl.BlockSpec((R, 1), lambda i: (i, 0)),
            pl.BlockSpec((D, D), lambda i: (0, 0)),
            pl.BlockSpec((D, D), lambda i: (0, 0)),
            pl.BlockSpec((D, D), lambda i: (0, 0)),
            pl.BlockSpec((D, D), lambda i: (0, 0)),
            pl.BlockSpec((1, D), lambda i: (0, 0)),
            pl.BlockSpec((1, D), lambda i: (0, 0)),
            pl.BlockSpec((1, D), lambda i: (0, 0)),
            pl.BlockSpec((1, D), lambda i: (0, 0)),
        ],
        out_specs=pl.BlockSpec((R, D), lambda i: (i, 0)),
        out_shape=jax.ShapeDtypeStruct((n, D), jnp.float32),
    )(px0, px1, pw, x_, m,
      W_c0[:, :D].T, W_c0[:, D:].T, W_c1[:, :D].T, W_c1[:, D:].T,
      b_c0.reshape(1, D), b_c1.reshape(1, D),
      gamma.reshape(1, D), beta.reshape(1, D))
    return out
